# baseline (speedup 1.0000x reference)
import functools

import jax
import jax.numpy as jnp
from jax import lax
from jax.experimental import pallas as pl
from jax.experimental.pallas import tpu as pltpu
from jax.experimental.pallas import tpu_sc as plsc

DIM = 128
NHOP = 2
KN = 32
HN = 50
HP = 128
RSW = 256
BN = 1024

NC = 2
NS = 16
NW = NC * NS
NCH = 2
BC = BN // NCH
PPW = BC // NW
PPH = BN // NW
NG = 128
NTR = PPW * KN // NG


def _schop_body(items_hbm, rs_hbm, ent_hbm, rel_hbm,
                o0_out, hrt_out,
                items_v, rsbuf, idx_all, obuf, gbuf0, gbuf1, gbuf2,
                sem_o, sem_rs, semg0, semg1, semg2, sems0, sems1, sems2):
    c = lax.axis_index("c")
    s = lax.axis_index("s")
    wid = s * NC + c
    base = wid * PPW

    pltpu.sync_copy(items_hbm.at[pl.ds(base, PPW)], items_v)
    cp_o = pltpu.async_copy(ent_hbm.at[items_v], obuf, sem_o)
    cp_rs = pltpu.async_copy(rs_hbm.at[items_v], rsbuf, sem_rs)

    cp_rs.wait()

    def build(p, carry):
        for slot in range(6):
            for chunk in range(KN // 16):
                v = rsbuf[p, pl.ds(slot * KN + chunk * 16, 16)]
                idx_all[slot, pl.ds(p * KN + chunk * 16, 16)] = v
        return carry
    lax.fori_loop(0, PPW, build, 0)

    cp_o.wait()
    pltpu.sync_copy(obuf, o0_out.at[pl.ds(base, PPW)])

    plan = []
    for slot in range(6):
        tab = rel_hbm if (slot % 3) == 1 else ent_hbm
        for j in range(NTR):
            plan.append((slot, j, tab))
    npl = len(plan)
    NB = 3
    gb = (gbuf0, gbuf1, gbuf2)
    gs = (semg0, semg1, semg2)
    ss = (sems0, sems1, sems2)

    def issue(i):
        slot, j, tab = plan[i]
        return pltpu.async_copy(
            tab.at[idx_all.at[slot, pl.ds(j * NG, NG)]], gb[i % NB],
            gs[i % NB])

    def store(i):
        slot, j, _ = plan[i]
        return pltpu.async_copy(
            gb[i % NB], hrt_out.at[slot, pl.ds(wid * PPW * KN + j * NG, NG)],
            ss[i % NB])

    gdesc = [issue(i) for i in range(NB)]
    sdesc = [None] * NB
    for i in range(npl):
        b = i % NB
        gdesc[b].wait()
        sdesc[b] = store(i)
        if i + NB < npl:
            sdesc[b].wait()
            gdesc[b] = issue(i + NB)
    for b in range(NB):
        if sdesc[b] is not None:
            sdesc[b].wait()


@functools.lru_cache(maxsize=1)
def _sc_hop():
  return functools.partial(
    pl.kernel,
    out_type=[
        jax.ShapeDtypeStruct((BC, DIM), jnp.float32),
        jax.ShapeDtypeStruct((6, BC * KN, DIM), jnp.float32)
    ],
    mesh=plsc.VectorSubcoreMesh(core_axis_name="c", subcore_axis_name="s",
                                num_cores=NC, num_subcores=NS),
    scratch_types=[
        pltpu.VMEM((PPW,), jnp.int32),
        pltpu.VMEM((PPW, RSW), jnp.int32),
        pltpu.VMEM((6, PPW * KN), jnp.int32),
        pltpu.VMEM((PPW, DIM), jnp.float32),
        pltpu.VMEM((NG, DIM), jnp.float32),
        pltpu.VMEM((NG, DIM), jnp.float32),
        pltpu.VMEM((NG, DIM), jnp.float32),
        pltpu.SemaphoreType.DMA,
        pltpu.SemaphoreType.DMA,
        pltpu.SemaphoreType.DMA,
        pltpu.SemaphoreType.DMA,
        pltpu.SemaphoreType.DMA,
        pltpu.SemaphoreType.DMA,
        pltpu.SemaphoreType.DMA,
        pltpu.SemaphoreType.DMA,
    ],
  )(_schop_body)


def _schist_body(users_hbm, hist_hbm, ent_hbm, user_out,
                 users_v, histbuf, hbuf0, hbuf1, uacc,
                 sem_h, semh0, semh1):
    c = lax.axis_index("c")
    s = lax.axis_index("s")
    wid = s * NC + c
    base = wid * PPH

    pltpu.sync_copy(users_hbm.at[pl.ds(base, PPH)], users_v)
    cp_h = pltpu.async_copy(hist_hbm.at[users_v], histbuf, sem_h)
    cp_h.wait()

    hb = (hbuf0, hbuf1)
    hs = (semh0, semh1)

    def hissue(p):
        return pltpu.async_copy(
            ent_hbm.at[histbuf.at[p, pl.ds(0, HN)]], hb[p % 2], hs[p % 2])

    hdesc = [hissue(0), hissue(1)]
    for p in range(PPH):
        hdesc[p % 2].wait()
        buf = hb[p % 2]

        def rsum(r, acc):
            return tuple(acc[cc] + buf[r, pl.ds(cc * 16, 16)]
                         for cc in range(DIM // 16))
        acc = lax.fori_loop(
            0, HN, rsum,
            tuple(jnp.zeros((16,), jnp.float32) for _ in range(DIM // 16)))
        for cc in range(DIM // 16):
            uacc[p, pl.ds(cc * 16, 16)] = acc[cc]
        if p + 2 < PPH:
            hdesc[p % 2] = hissue(p + 2)
    pltpu.sync_copy(uacc, user_out.at[pl.ds(base, PPH)])


@functools.lru_cache(maxsize=1)
def _sc_hist():
  return functools.partial(
    pl.kernel,
    out_type=jax.ShapeDtypeStruct((BN, DIM), jnp.float32),
    mesh=plsc.VectorSubcoreMesh(core_axis_name="c", subcore_axis_name="s",
                                num_cores=NC, num_subcores=NS),
    scratch_types=[
        pltpu.VMEM((PPH,), jnp.int32),
        pltpu.VMEM((PPH, HP), jnp.int32),
        pltpu.VMEM((HN, DIM), jnp.float32),
        pltpu.VMEM((HN, DIM), jnp.float32),
        pltpu.VMEM((PPH, DIM), jnp.float32),
        pltpu.SemaphoreType.DMA,
        pltpu.SemaphoreType.DMA,
        pltpu.SemaphoreType.DMA,
    ],
  )(_schist_body)


def _fmt_hist_body(hist_ref, histp_ref):
    ht = jnp.transpose(hist_ref[...])
    histp_ref[...] = jnp.concatenate(
        [ht, jnp.zeros((ht.shape[0], HP - HN), jnp.int32)], axis=1)


def _fmt_rs_body(rs_ref, rsp_ref):
    rt = jnp.transpose(rs_ref[...])
    rsp_ref[...] = jnp.concatenate(
        [rt, jnp.zeros((rt.shape[0], RSW - NHOP * 3 * KN), jnp.int32)], axis=1)


def _fmt_hist_call(hist_t):
    nh = hist_t.shape[1]
    hb = 1024
    return pl.pallas_call(
        _fmt_hist_body,
        grid=(10,),
        in_specs=[pl.BlockSpec((HN, hb), lambda g: (0, g))],
        out_specs=pl.BlockSpec((hb, HP), lambda g: (g, 0)),
        out_shape=jax.ShapeDtypeStruct((nh, HP), jnp.int32),
    )(hist_t)


def _fmt_rs_call(rsf_t):
    nr = rsf_t.shape[1]
    rb = 2048
    return pl.pallas_call(
        _fmt_rs_body,
        grid=(10,),
        in_specs=[pl.BlockSpec((NHOP * 3 * KN, rb), lambda g: (0, g))],
        out_specs=pl.BlockSpec((rb, RSW), lambda g: (g, 0)),
        out_shape=jax.ShapeDtypeStruct((nr, RSW), jnp.int32),
    )(rsf_t)


PB = 128
GRID = BC // PB


def _tc_body(hrt, o0, wih, whh, b2, acc_ref):
    acc = o0[...]
    b = b2[...]
    wihv = wih[...]
    whhv = whh[...]
    for hop in range(NHOP):
        hd = hrt[3 * hop]
        rl = hrt[3 * hop + 1]
        tl = hrt[3 * hop + 2]
        hr = jnp.concatenate([hd, rl], axis=1)
        tr = jnp.concatenate([tl, rl], axis=1)
        nt = (((1,), (1,)), ((), ()))
        x1w = lax.dot_general(hr, wihv, nt, preferred_element_type=jnp.float32)
        x2w = lax.dot_general(tr, wihv, nt, preferred_element_type=jnp.float32)
        h1v = jnp.maximum(x1w + b, 0.0)
        h2v = jnp.maximum(
            x2w + lax.dot_general(h1v, whhv, nt,
                                  preferred_element_type=jnp.float32) + b, 0.0)
        lcol = jnp.sum(hr * tr, axis=1, keepdims=True)
        l3 = lcol.reshape(PB, KN, 1)
        m3 = jnp.max(l3, axis=1, keepdims=True)
        e3 = jnp.exp(l3 - m3)
        d3 = jnp.sum(e3, axis=1, keepdims=True)
        picol = (e3 / d3).reshape(PB * KN, 1)
        acc = acc + jnp.sum((h2v * picol).reshape(PB, KN, DIM), axis=1)
    acc_ref[...] = acc


def _tc_call(hrt, o0, wih, whh, b2):
    pairs = pl.BlockSpec((PB, DIM), lambda g: (g, 0))
    return pl.pallas_call(
        _tc_body,
        grid=(GRID,),
        in_specs=[pl.BlockSpec((6, PB * KN, DIM), lambda g: (0, g, 0)),
                  pairs,
                  pl.BlockSpec((DIM, 2 * DIM), lambda g: (0, 0)),
                  pl.BlockSpec((DIM, DIM), lambda g: (0, 0)),
                  pl.BlockSpec((1, DIM), lambda g: (0, 0))],
        out_specs=pairs,
        out_shape=jax.ShapeDtypeStruct((BC, DIM), jnp.float32),
    )(hrt, o0, wih, whh, b2)


def _tcf_body(a0, a1, ue, out_ref):
    sel = pl.program_id(0) < GRID
    acc = jnp.where(sel, a0[...], a1[...])
    logit = jnp.sum(ue[...] * acc, axis=1, keepdims=True)
    out_ref[...] = 1.0 / (1.0 + jnp.exp(-logit))


def _tcf_call(acc0, acc1, ue):
    return pl.pallas_call(
        _tcf_body,
        grid=(BN // PB,),
        in_specs=[
            pl.BlockSpec((PB, DIM),
                         lambda g: (jnp.minimum(g, GRID - 1), 0)),
            pl.BlockSpec((PB, DIM),
                         lambda g: (jnp.clip(g - GRID, 0, GRID - 1), 0)),
            pl.BlockSpec((PB, DIM), lambda g: (g, 0)),
        ],
        out_specs=pl.BlockSpec((PB, 1), lambda g: (g, 0)),
        out_shape=jax.ShapeDtypeStruct((BN, 1), jnp.float32),
    )(acc0, acc1, ue)


def kernel(pairs, history_dict, ripple_sets, entity_embedding_mat,
           relation_embedding_mat, W_ih, W_hh, b_ih, b_hh):
    users = pairs[:, 0]
    items = pairs[:, 1]
    hist_p = _fmt_hist_call(history_dict.T)
    rs_flat = _fmt_rs_call(
        ripple_sets.reshape(ripple_sets.shape[0], NHOP * 3 * KN).T)
    b2 = (b_ih + b_hh).reshape(1, DIM)
    hop = _sc_hop()
    accs = []
    for ch in range(NCH):
        o0, hrt = hop(lax.slice(items, (ch * BC,), ((ch + 1) * BC,)),
                      rs_flat, entity_embedding_mat, relation_embedding_mat)
        accs.append(_tc_call(hrt, o0, W_ih, W_hh, b2))
    ue = _sc_hist()(users, hist_p, entity_embedding_mat)
    return _tcf_call(accs[0], accs[1], ue).reshape(BN)


# speedup vs baseline: 1.0247x; 1.0247x over previous
import functools

import jax
import jax.numpy as jnp
from jax import lax
from jax.experimental import pallas as pl
from jax.experimental.pallas import tpu as pltpu
from jax.experimental.pallas import tpu_sc as plsc

DIM = 128
NHOP = 2
KN = 32
HN = 50
HP = 128
RSW = 256
BN = 1024

NC = 2
NS = 16
NW = NC * NS
NCH = 2
BC = BN // NCH
PPW = BC // NW
NG = 128
NTR = PPW * KN // NG


def _sc_body(users_hbm, items_hbm, hist_hbm, rs_hbm, ent_hbm, rel_hbm,
             user_out, o0_out, hrt_out,
             users_v, items_v, rsbuf, idx_all, histbuf,
             hbuf0, hbuf1, uacc, obuf, gbuf0, gbuf1, gbuf2,
             sem_o, sem_rs, sem_h, semg0, semg1, semg2, semh0, semh1):
    c = lax.axis_index("c")
    s = lax.axis_index("s")
    wid = s * NC + c
    base = wid * PPW

    pltpu.sync_copy(users_hbm.at[pl.ds(base, PPW)], users_v)
    pltpu.sync_copy(items_hbm.at[pl.ds(base, PPW)], items_v)
    cp_o = pltpu.async_copy(ent_hbm.at[items_v], obuf, sem_o)
    cp_rs = pltpu.async_copy(rs_hbm.at[items_v], rsbuf, sem_rs)
    cp_h = pltpu.async_copy(hist_hbm.at[users_v], histbuf, sem_h)

    cp_rs.wait()

    def build(p, carry):
        for slot in range(6):
            for chunk in range(KN // 16):
                v = rsbuf[p, pl.ds(slot * KN + chunk * 16, 16)]
                idx_all[slot, pl.ds(p * KN + chunk * 16, 16)] = v
        return carry
    lax.fori_loop(0, PPW, build, 0)

    cp_o.wait()
    pltpu.sync_copy(obuf, o0_out.at[pl.ds(base, PPW)])

    plan = []
    for slot in range(6):
        tab = rel_hbm if (slot % 3) == 1 else ent_hbm
        for j in range(NTR):
            plan.append((slot, j, tab))
    npl = len(plan)
    NB = 3
    gb = (gbuf0, gbuf1, gbuf2)
    gs = (semg0, semg1, semg2)
    hb = (hbuf0, hbuf1)
    hs = (semh0, semh1)

    def issue(i):
        slot, j, tab = plan[i]
        return pltpu.async_copy(
            tab.at[idx_all.at[slot, pl.ds(j * NG, NG)]], gb[i % NB],
            gs[i % NB])

    def hissue(p):
        return pltpu.async_copy(
            ent_hbm.at[histbuf.at[p, pl.ds(0, HN)]], hb[p % 2], hs[p % 2])

    def hist_pair(p):
        buf = hb[p % 2]

        def rsum(r, acc):
            return tuple(acc[cc] + buf[r, pl.ds(cc * 16, 16)]
                         for cc in range(DIM // 16))
        acc = lax.fori_loop(
            0, HN, rsum,
            tuple(jnp.zeros((16,), jnp.float32) for _ in range(DIM // 16)))
        for cc in range(DIM // 16):
            uacc[p, pl.ds(cc * 16, 16)] = acc[cc]

    gdesc = [issue(i) for i in range(NB)]
    cp_h.wait()
    hdesc = [hissue(0), hissue(1) if PPW > 1 else None]
    for i in range(npl):
        b = i % NB
        gdesc[b].wait()
        slot, j, _ = plan[i]
        pltpu.sync_copy(gb[b],
                        hrt_out.at[slot, pl.ds(wid * PPW * KN + j * NG, NG)])
        if i < PPW:
            hdesc[i % 2].wait()
            hist_pair(i)
            if i + 2 < PPW:
                hdesc[i % 2] = hissue(i + 2)
        if i + NB < npl:
            gdesc[b] = issue(i + NB)
    pltpu.sync_copy(uacc, user_out.at[pl.ds(base, PPW)])


@functools.lru_cache(maxsize=1)
def _sc_gather():
  return functools.partial(
    pl.kernel,
    out_type=[
        jax.ShapeDtypeStruct((BC, DIM), jnp.float32),
        jax.ShapeDtypeStruct((BC, DIM), jnp.float32),
        jax.ShapeDtypeStruct((6, BC * KN, DIM), jnp.float32)
    ],
    mesh=plsc.VectorSubcoreMesh(core_axis_name="c", subcore_axis_name="s",
                                num_cores=NC, num_subcores=NS),
    scratch_types=[
        pltpu.VMEM((PPW,), jnp.int32),
        pltpu.VMEM((PPW,), jnp.int32),
        pltpu.VMEM((PPW, RSW), jnp.int32),
        pltpu.VMEM((6, PPW * KN), jnp.int32),
        pltpu.VMEM((PPW, HP), jnp.int32),
        pltpu.VMEM((HN, DIM), jnp.float32),
        pltpu.VMEM((HN, DIM), jnp.float32),
        pltpu.VMEM((PPW, DIM), jnp.float32),
        pltpu.VMEM((PPW, DIM), jnp.float32),
        pltpu.VMEM((NG, DIM), jnp.float32),
        pltpu.VMEM((NG, DIM), jnp.float32),
        pltpu.VMEM((NG, DIM), jnp.float32),
        pltpu.SemaphoreType.DMA,
        pltpu.SemaphoreType.DMA,
        pltpu.SemaphoreType.DMA,
        pltpu.SemaphoreType.DMA,
        pltpu.SemaphoreType.DMA,
        pltpu.SemaphoreType.DMA,
        pltpu.SemaphoreType.DMA,
        pltpu.SemaphoreType.DMA,
    ],
  )(_sc_body)


def _fmt_body(hist_ref, rs_ref, histp_ref, rsp_ref):
    ht = jnp.transpose(hist_ref[...])
    histp_ref[...] = jnp.concatenate(
        [ht, jnp.zeros((ht.shape[0], HP - HN), jnp.int32)], axis=1)
    rt = jnp.transpose(rs_ref[...])
    rsp_ref[...] = jnp.concatenate(
        [rt, jnp.zeros((rt.shape[0], RSW - NHOP * 3 * KN), jnp.int32)], axis=1)


def _fmt_call(hist_t, rsf_t):
    nh, nr = hist_t.shape[1], rsf_t.shape[1]
    hb, rb = 1024, 2048
    return pl.pallas_call(
        _fmt_body,
        grid=(10,),
        in_specs=[pl.BlockSpec((HN, hb), lambda g: (0, g)),
                  pl.BlockSpec((NHOP * 3 * KN, rb), lambda g: (0, g))],
        out_specs=[pl.BlockSpec((hb, HP), lambda g: (g, 0)),
                   pl.BlockSpec((rb, RSW), lambda g: (g, 0))],
        out_shape=[jax.ShapeDtypeStruct((nh, HP), jnp.int32),
                   jax.ShapeDtypeStruct((nr, RSW), jnp.int32)],
    )(hist_t, rsf_t)


PB = 128
GRID = BC // PB


def _tc_body(hrt, o0, ue, wih, whh, b2, out_ref):
    acc = o0[...]
    b = b2[...]
    wihv = wih[...]
    whhv = whh[...]
    for hop in range(NHOP):
        hd = hrt[3 * hop]
        rl = hrt[3 * hop + 1]
        tl = hrt[3 * hop + 2]
        hr = jnp.concatenate([hd, rl], axis=1)
        tr = jnp.concatenate([tl, rl], axis=1)
        nt = (((1,), (1,)), ((), ()))
        x1w = lax.dot_general(hr, wihv, nt, preferred_element_type=jnp.float32)
        x2w = lax.dot_general(tr, wihv, nt, preferred_element_type=jnp.float32)
        h1v = jnp.maximum(x1w + b, 0.0)
        h2v = jnp.maximum(
            x2w + lax.dot_general(h1v, whhv, nt,
                                  preferred_element_type=jnp.float32) + b, 0.0)
        lcol = jnp.sum(hr * tr, axis=1, keepdims=True)
        l3 = lcol.reshape(PB, KN, 1)
        m3 = jnp.max(l3, axis=1, keepdims=True)
        e3 = jnp.exp(l3 - m3)
        d3 = jnp.sum(e3, axis=1, keepdims=True)
        picol = (e3 / d3).reshape(PB * KN, 1)
        acc = acc + jnp.sum((h2v * picol).reshape(PB, KN, DIM), axis=1)
    logit = jnp.sum(ue[...] * acc, axis=1, keepdims=True)
    out_ref[...] = 1.0 / (1.0 + jnp.exp(-logit))


def _tc_call(hrt, o0, ue, wih, whh, b2):
    pairs = pl.BlockSpec((PB, DIM), lambda g: (g, 0))
    return pl.pallas_call(
        _tc_body,
        grid=(GRID,),
        in_specs=[pl.BlockSpec((6, PB * KN, DIM), lambda g: (0, g, 0)),
                  pairs, pairs,
                  pl.BlockSpec((DIM, 2 * DIM), lambda g: (0, 0)),
                  pl.BlockSpec((DIM, DIM), lambda g: (0, 0)),
                  pl.BlockSpec((1, DIM), lambda g: (0, 0))],
        out_specs=pl.BlockSpec((PB, 1), lambda g: (g, 0)),
        out_shape=jax.ShapeDtypeStruct((BC, 1), jnp.float32),
    )(hrt, o0, ue, wih, whh, b2)


def kernel(pairs, history_dict, ripple_sets, entity_embedding_mat,
           relation_embedding_mat, W_ih, W_hh, b_ih, b_hh):
    users = pairs[:, 0]
    items = pairs[:, 1]
    hist_p, rs_flat = _fmt_call(
        history_dict.T,
        ripple_sets.reshape(ripple_sets.shape[0], NHOP * 3 * KN).T)
    b2 = (b_ih + b_hh).reshape(1, DIM)
    sc = _sc_gather()
    staged = [sc(lax.slice(users, (ch * BC,), ((ch + 1) * BC,)),
                 lax.slice(items, (ch * BC,), ((ch + 1) * BC,)),
                 hist_p, rs_flat,
                 entity_embedding_mat, relation_embedding_mat)
              for ch in range(NCH)]
    outs = [_tc_call(hrt, o0, user_emb, W_ih, W_hh, b2)
            for user_emb, o0, hrt in staged]
    return jnp.concatenate(outs, axis=0).reshape(BN)
